# baseline (speedup 1.0000x reference)
import functools

import jax
import jax.numpy as jnp
from jax import lax
from jax.experimental import pallas as pl
from jax.experimental.pallas import tpu as pltpu


def _round_up(x, m):
    return (x + m - 1) // m * m


_DOT_DIMS_KN = (((1,), (0,)), ((), ()))


def _linear_silu_kernel(x_ref, w_ref, b_ref, o_ref):
    z = lax.dot_general(
        x_ref[...].astype(jnp.bfloat16), w_ref[...],
        dimension_numbers=_DOT_DIMS_KN,
        preferred_element_type=jnp.float32,
    )
    z = z + b_ref[...]
    o_ref[...] = (z * jax.nn.sigmoid(z)).astype(o_ref.dtype)


@functools.partial(jax.jit, static_argnames=("tm", "nsplit"))
def _linear_silu(x, weight, bias, *, tm=128, nsplit=1):
    orig_shape = x.shape
    K = orig_shape[-1]
    N, Kw = weight.shape
    assert Kw == K

    M = 1
    for d in orig_shape[:-1]:
        M *= d
    x2d = x.reshape(M, K)
    b2d = bias.astype(jnp.float32).reshape(1, N)

    tn = N // nsplit
    Mp = _round_up(M, tm)
    xb = x2d
    wt = weight.astype(jnp.bfloat16).T
    if Mp != M:
        xb = jnp.pad(xb, ((0, Mp - M), (0, 0)))

    nm = Mp // tm

    cost = pl.CostEstimate(
        flops=2 * M * N * K,
        transcendentals=M * N,
        bytes_accessed=M * K * 4 * nsplit + N * K * 2 + (N + M * N) * 4,
    )

    out = pl.pallas_call(
        _linear_silu_kernel,
        out_shape=jax.ShapeDtypeStruct((Mp, N), x.dtype),
        grid=(nsplit, nm),
        in_specs=[
            pl.BlockSpec((tm, K), lambda j, i: (i, 0)),
            pl.BlockSpec((K, tn), lambda j, i: (0, j)),
            pl.BlockSpec((1, tn), lambda j, i: (0, j)),
        ],
        out_specs=pl.BlockSpec((tm, tn), lambda j, i: (i, j)),
        compiler_params=pltpu.CompilerParams(
            dimension_semantics=("parallel", "arbitrary")
        ),
        cost_estimate=cost,
    )(xb, wt, b2d)

    if Mp != M:
        out = out[:M]
    return out.reshape(*orig_shape[:-1], N)


def kernel(x, weight, bias):
    return _linear_silu(x, weight, bias)


# speedup vs baseline: 1.0497x; 1.0497x over previous
import functools

import jax
import jax.numpy as jnp
from jax import lax
from jax.experimental import pallas as pl
from jax.experimental.pallas import tpu as pltpu


def _round_up(x, m):
    return (x + m - 1) // m * m


_DOT_DIMS = (((1,), (1,)), ((), ()))


def _linear_silu_kernel(x_ref, w_ref, b_ref, o_ref):
    z = lax.dot_general(
        x_ref[...].astype(jnp.bfloat16), w_ref[...],
        dimension_numbers=_DOT_DIMS,
        preferred_element_type=jnp.float32,
    )
    z = z + b_ref[...]
    o_ref[...] = (z * jax.nn.sigmoid(z)).astype(o_ref.dtype)


@functools.partial(jax.jit, static_argnames=("tm",))
def _linear_silu(x, weight, bias, *, tm=256):
    orig_shape = x.shape
    K = orig_shape[-1]
    N, Kw = weight.shape
    assert Kw == K

    M = 1
    for d in orig_shape[:-1]:
        M *= d
    x2d = x.reshape(M, K)
    b2d = bias.astype(jnp.float32).reshape(1, N)

    tm = min(tm, _round_up(M, 8))
    Mp = _round_up(M, tm)
    xb = x2d
    wb = weight.astype(jnp.bfloat16)
    if Mp != M:
        xb = jnp.pad(xb, ((0, Mp - M), (0, 0)))

    nm = Mp // tm

    cost = pl.CostEstimate(
        flops=2 * M * N * K,
        transcendentals=M * N,
        bytes_accessed=M * K * 4 + N * K * 2 + (N + M * N) * 4,
    )

    out = pl.pallas_call(
        _linear_silu_kernel,
        out_shape=jax.ShapeDtypeStruct((Mp, N), x.dtype),
        grid=(nm,),
        in_specs=[
            pl.BlockSpec((tm, K), lambda i: (i, 0)),
            pl.BlockSpec((N, K), lambda i: (0, 0)),
            pl.BlockSpec((1, N), lambda i: (0, 0)),
        ],
        out_specs=pl.BlockSpec((tm, N), lambda i: (i, 0)),
        compiler_params=pltpu.CompilerParams(
            dimension_semantics=("parallel",)
        ),
        cost_estimate=cost,
    )(xb, wb, b2d)

    if Mp != M:
        out = out[:M]
    return out.reshape(*orig_shape[:-1], N)


def kernel(x, weight, bias):
    return _linear_silu(x, weight, bias)


# speedup vs baseline: 1.0522x; 1.0024x over previous
import functools

import jax
import jax.numpy as jnp
from jax import lax
from jax.experimental import pallas as pl
from jax.experimental.pallas import tpu as pltpu


def _round_up(x, m):
    return (x + m - 1) // m * m


_DOT_DIMS = (((1,), (1,)), ((), ()))


def _linear_silu_kernel(x_ref, w_ref, b_ref, o_ref):
    z = lax.dot_general(
        x_ref[...].astype(jnp.bfloat16), w_ref[...],
        dimension_numbers=_DOT_DIMS,
        preferred_element_type=jnp.float32,
    )
    z = z + b_ref[...]
    o_ref[...] = (z * jax.nn.sigmoid(z)).astype(o_ref.dtype)


@functools.partial(jax.jit, static_argnames=("tm",))
def _linear_silu(x, weight, bias, *, tm=256):
    orig_shape = x.shape
    K = orig_shape[-1]
    N, Kw = weight.shape
    assert Kw == K

    M = 1
    for d in orig_shape[:-1]:
        M *= d
    x2d = x.reshape(M, K)
    b2d = bias.astype(jnp.float32).reshape(1, N)

    tm = min(tm, _round_up(M, 8))
    Mp = _round_up(M, tm)
    xb = x2d
    wb = weight.astype(jnp.bfloat16)
    if Mp != M:
        xb = jnp.pad(xb, ((0, Mp - M), (0, 0)))

    nm = Mp // tm

    cost = pl.CostEstimate(
        flops=2 * M * N * K,
        transcendentals=M * N,
        bytes_accessed=M * K * 4 + N * K * 2 + (N + M * N) * 4,
    )

    out = pl.pallas_call(
        _linear_silu_kernel,
        out_shape=jax.ShapeDtypeStruct((Mp, N), x.dtype),
        grid=(nm,),
        in_specs=[
            pl.BlockSpec((tm, K), lambda i: (i, 0)),
            pl.BlockSpec((N, K), lambda i: (0, 0),
                         pipeline_mode=pl.Buffered(buffer_count=1)),
            pl.BlockSpec((1, N), lambda i: (0, 0)),
        ],
        out_specs=pl.BlockSpec((tm, N), lambda i: (i, 0)),
        compiler_params=pltpu.CompilerParams(
            dimension_semantics=("parallel",)
        ),
        cost_estimate=cost,
    )(xb, wb, b2d)

    if Mp != M:
        out = out[:M]
    return out.reshape(*orig_shape[:-1], N)


def kernel(x, weight, bias):
    return _linear_silu(x, weight, bias)
